# baseline (speedup 1.0000x reference)
import functools

import jax
import jax.numpy as jnp
from jax import lax
from jax.experimental import pallas as pl
from jax.experimental.pallas import tpu as pltpu
from jax.experimental.pallas import tpu_sc as plsc

D_MODEL = 64
N_TOKENS = 4096 * 200
NUM_CORES = 2
NUM_SUBCORES = 16
NUM_WORKERS = NUM_CORES * NUM_SUBCORES
PER_WORKER = N_TOKENS // NUM_WORKERS
CHUNK = 800
NUM_CHUNKS = PER_WORKER // CHUNK

_mesh = plsc.VectorSubcoreMesh(
    core_axis_name="c", subcore_axis_name="s")


@functools.partial(
    pl.kernel,
    out_type=jax.ShapeDtypeStruct((N_TOKENS, D_MODEL), jnp.float32),
    mesh=_mesh,
    scratch_types=[
        pltpu.VMEM((CHUNK,), jnp.int32),
        pltpu.VMEM((CHUNK, D_MODEL), jnp.float32),
        pltpu.SemaphoreType.DMA,
    ],
    compiler_params=pltpu.CompilerParams(use_tc_tiling_on_sc=False),
)
def _embed_gather(tok_hbm, table_hbm, out_hbm, idx_v, rows_v, sem):
    wid = lax.axis_index("s") * NUM_CORES + lax.axis_index("c")
    base = wid * PER_WORKER

    def body(g, carry):
        off = base + g * CHUNK
        pltpu.sync_copy(tok_hbm.at[pl.ds(off, CHUNK)], idx_v)
        pltpu.async_copy(table_hbm.at[idx_v], rows_v, sem).wait()
        pltpu.sync_copy(rows_v, out_hbm.at[pl.ds(off, CHUNK)])
        return carry

    lax.fori_loop(0, NUM_CHUNKS, body, 0)


def kernel(tokens, param):
    flat = tokens.reshape(-1)
    out = _embed_gather(flat, param)
    return out.reshape(tokens.shape + (param.shape[-1],))


# speedup vs baseline: 1.0164x; 1.0164x over previous
import functools

import jax
import jax.numpy as jnp
from jax import lax
from jax.experimental import pallas as pl
from jax.experimental.pallas import tpu as pltpu
from jax.experimental.pallas import tpu_sc as plsc

D_MODEL = 64
N_TOKENS = 4096 * 200
NUM_CORES = 2
NUM_SUBCORES = 16
NUM_WORKERS = NUM_CORES * NUM_SUBCORES
PER_WORKER = N_TOKENS // NUM_WORKERS
CHUNK = 800
NUM_CHUNKS = PER_WORKER // CHUNK
NUM_PAIRS = NUM_CHUNKS // 2

_mesh = plsc.VectorSubcoreMesh(
    core_axis_name="c", subcore_axis_name="s")


@functools.partial(
    pl.kernel,
    out_type=jax.ShapeDtypeStruct((N_TOKENS, D_MODEL), jnp.float32),
    mesh=_mesh,
    scratch_types=[
        pltpu.VMEM((CHUNK,), jnp.int32),
        pltpu.VMEM((CHUNK,), jnp.int32),
        pltpu.VMEM((CHUNK, D_MODEL), jnp.float32),
        pltpu.VMEM((CHUNK, D_MODEL), jnp.float32),
        pltpu.SemaphoreType.DMA,
        pltpu.SemaphoreType.DMA,
        pltpu.SemaphoreType.DMA,
        pltpu.SemaphoreType.DMA,
    ],
    compiler_params=pltpu.CompilerParams(use_tc_tiling_on_sc=False),
)
def _embed_gather(tok_hbm, table_hbm, out_hbm, idx0, idx1, rows0, rows1,
                  sg0, sg1, sw0, sw1):
    wid = lax.axis_index("s") * NUM_CORES + lax.axis_index("c")
    base = wid * PER_WORKER

    def out_at(g):
        return out_hbm.at[pl.ds(base + g * CHUNK, CHUNK)]

    pltpu.sync_copy(tok_hbm.at[pl.ds(base, CHUNK)], idx0)
    pltpu.async_copy(table_hbm.at[idx0], rows0, sg0)

    def body(i, carry):
        g0 = 2 * i
        g1 = g0 + 1
        @pl.when(i > 0)
        def _():
            pltpu.make_async_copy(rows1, out_at(g0 - 1), sw1).wait()
        pltpu.sync_copy(tok_hbm.at[pl.ds(base + g1 * CHUNK, CHUNK)], idx1)
        pltpu.async_copy(table_hbm.at[idx1], rows1, sg1)
        pltpu.make_async_copy(table_hbm.at[idx0], rows0, sg0).wait()
        pltpu.async_copy(rows0, out_at(g0), sw0)
        @pl.when(i + 1 < NUM_PAIRS)
        def _():
            pltpu.make_async_copy(rows0, out_at(g0), sw0).wait()
            pltpu.sync_copy(
                tok_hbm.at[pl.ds(base + (g0 + 2) * CHUNK, CHUNK)], idx0)
            pltpu.async_copy(table_hbm.at[idx0], rows0, sg0)
        pltpu.make_async_copy(table_hbm.at[idx1], rows1, sg1).wait()
        pltpu.async_copy(rows1, out_at(g1), sw1)
        return carry

    lax.fori_loop(0, NUM_PAIRS, body, 0)
    pltpu.make_async_copy(rows0, out_at(NUM_CHUNKS - 2), sw0).wait()
    pltpu.make_async_copy(rows1, out_at(NUM_CHUNKS - 1), sw1).wait()


def kernel(tokens, param):
    flat = tokens.reshape(-1)
    out = _embed_gather(flat, param)
    return out.reshape(tokens.shape + (param.shape[-1],))
